# baseline (speedup 1.0000x reference)
import functools

import jax
import jax.numpy as jnp
from jax import lax
from jax.experimental import pallas as pl
from jax.experimental.pallas import tpu as pltpu
from jax.experimental.pallas import tpu_sc as plsc

NC, NS, LANES = 2, 16, 16

LCH = 20
DW = 2


def _make_k(Ldim, DT, BT, D8, B128):
    npairs = BT // NC
    nch = Ldim // LCH
    mesh = plsc.VectorSubcoreMesh(core_axis_name="c", subcore_axis_name="s")
    scratch = [
        pltpu.VMEM((Ldim, DW, B128), jnp.float32),
        pltpu.VMEM((Ldim, DW, B128), jnp.float32),
        pltpu.VMEM((LCH, B128), jnp.int16),
        pltpu.VMEM((LCH, B128), jnp.int16),
        pltpu.VMEM((LCH, DW, B128), jnp.float32),
        pltpu.VMEM((LCH, DW, B128), jnp.float32),
        pltpu.SemaphoreType.DMA,
        pltpu.SemaphoreType.DMA,
        pltpu.SemaphoreType.DMA,
        pltpu.SemaphoreType.DMA,
        pltpu.SemaphoreType.DMA,
        pltpu.SemaphoreType.DMA,
    ]

    @functools.partial(
        pl.kernel,
        out_type=jax.ShapeDtypeStruct((Ldim, DT, BT, D8, B128), jnp.float32),
        mesh=mesh,
        scratch_types=scratch,
        compiler_params=pltpu.CompilerParams(
            use_tc_tiling_on_sc=False, needs_layout_passes=False),
    )
    def k(rt5, it2, out5, stgA, stgB, idxA, idxB, outA, outB,
          stgAs, stgBs, idxAs, idxBs, outAs, outBs):
        cc = lax.axis_index("c")
        ss = lax.axis_index("s")
        dt = ss // 2
        spair = ss % 2
        iota = lax.iota(jnp.int32, LANES)
        bvecs = [j * LANES + iota for j in range(B128 // LANES)]
        djvecs = [jnp.zeros((LANES,), jnp.int32) + dj for dj in range(DW)]

        def stg_src(pair, d80):
            bt = cc * npairs + pair
            return rt5.at[:, dt, bt, pl.ds(d80, DW), :]

        def idx_src(bt, chunk_l0):
            return it2.at[pl.ds(chunk_l0, LCH), pl.ds(bt * B128, B128)]

        def out_dst(bt, d80, l0):
            return out5.at[pl.ds(l0, LCH), dt, bt, pl.ds(d80, DW), :]

        def compute_chunk(stg, idxbuf, outbuf):
            nj = B128 // LANES

            @plsc.parallel_loop(0, LCH, step=1, unroll=8)
            def _body(li):
                gs = {}
                for m in range(nj // 2):
                    packed = plsc.bitcast(
                        idxbuf[li, pl.ds(m * 2 * LANES, 2 * LANES)], jnp.int32)
                    gs[2 * m] = packed & 0xFFFF
                    gs[2 * m + 1] = lax.shift_right_logical(packed, 16)
                vals = [(j, dj,
                         plsc.load_gather(stg, [gs[j], djvecs[dj], bvecs[j]]))
                        for j in range(nj) for dj in range(DW)]
                for j, dj, v in vals:
                    outbuf[li, dj, pl.ds(j * LANES, LANES)] = v

        def unit_compute(pair, bt, d80, stg, next_bt, next_valid, is_first_unit):
            def body(t, carry):
                for kpar, (idxbuf, isem, outbuf, osem) in enumerate(
                        [(idxA, idxAs, outA, outAs), (idxB, idxBs, outB, outBs)]):
                    kchunk = 2 * t + kpar
                    l0 = kchunk * LCH
                    pltpu.make_async_copy(idx_src(bt, l0), idxbuf, isem).wait()
                    if is_first_unit:
                        skip = jnp.logical_and(pair == 0, t == 0)
                    else:
                        skip = jnp.logical_and(pair < 0, t == 0)

                    @pl.when(jnp.logical_not(skip))
                    def _():
                        pltpu.make_async_copy(outbuf, out_dst(bt, d80, l0), osem).wait()

                    compute_chunk(stg, idxbuf, outbuf)
                    pltpu.async_copy(outbuf, out_dst(bt, d80, l0), osem)

                    @pl.when(t < nch // 2 - 1)
                    def _():
                        pltpu.async_copy(idx_src(bt, l0 + 2 * LCH), idxbuf, isem)

                    @pl.when(jnp.logical_and(t == nch // 2 - 1, next_valid))
                    def _():
                        pltpu.async_copy(idx_src(next_bt, kpar * LCH), idxbuf, isem)
                return carry

            lax.fori_loop(0, nch // 2, body, 0)

        d8A = spair * 2 * DW
        d8B = (spair * 2 + 1) * DW
        pltpu.async_copy(stg_src(0, d8A), stgA, stgAs)
        pltpu.async_copy(stg_src(0, d8B), stgB, stgBs)
        bt0 = cc * npairs
        pltpu.async_copy(idx_src(bt0, 0), idxA, idxAs)
        pltpu.async_copy(idx_src(bt0, LCH), idxB, idxBs)

        def pair_body(pair, carry):
            bt = cc * npairs + pair
            has_next = pair + 1 < npairs
            pltpu.make_async_copy(stg_src(pair, d8A), stgA, stgAs).wait()
            unit_compute(pair, bt, d8A, stgA,
                         next_bt=bt, next_valid=jnp.bool_(True),
                         is_first_unit=True)

            @pl.when(has_next)
            def _():
                pltpu.async_copy(stg_src(pair + 1, d8A), stgA, stgAs)

            pltpu.make_async_copy(stg_src(pair, d8B), stgB, stgBs).wait()
            unit_compute(pair, bt, d8B, stgB,
                         next_bt=bt + 1, next_valid=has_next,
                         is_first_unit=False)

            @pl.when(has_next)
            def _():
                pltpu.async_copy(stg_src(pair + 1, d8B), stgB, stgBs)

            return carry

        lax.fori_loop(0, npairs, pair_body, 0)

        lastb = cc * npairs + npairs - 1
        l_last = (nch - 2) * LCH
        pltpu.make_async_copy(outA, out_dst(lastb, d8B, l_last), outAs).wait()
        pltpu.make_async_copy(outB, out_dst(lastb, d8B, l_last + LCH), outBs).wait()

    return k


def kernel(reps, index):
    B, L, D = reps.shape
    DT, D8, B128 = D // 8, 8, 128
    BT = B // B128
    rt = jnp.transpose(reps, (1, 2, 0))
    rt5 = rt.reshape(L, DT, D8, BT, B128)
    rt5 = jnp.transpose(rt5, (0, 1, 3, 2, 4))
    it2 = jnp.transpose(index.reshape(B, L).astype(jnp.int16), (1, 0))
    it2p = (it2.reshape(L, B // 32, 2, LANES)
            .transpose(0, 1, 3, 2)
            .reshape(L, B))
    out5 = _make_k(L, DT, BT, D8, B128)(rt5, it2p)
    out = jnp.transpose(out5, (0, 1, 3, 2, 4)).reshape(L, D, B)
    return jnp.transpose(out, (2, 0, 1))


# speedup vs baseline: 1.0512x; 1.0512x over previous
import functools

import jax
import jax.numpy as jnp
from jax import lax
from jax.experimental import pallas as pl
from jax.experimental.pallas import tpu as pltpu
from jax.experimental.pallas import tpu_sc as plsc

NC, NS, LANES = 2, 16, 16

LCH = 25
DW = 2


def _make_k(Ldim, DT, BT, D8, B128):
    npairs = BT // NC
    nch = Ldim // LCH
    mesh = plsc.VectorSubcoreMesh(core_axis_name="c", subcore_axis_name="s")
    scratch = [
        pltpu.VMEM((Ldim, DW, B128), jnp.float32),
        pltpu.VMEM((Ldim, DW, B128), jnp.float32),
        pltpu.VMEM((LCH, B128), jnp.int16),
        pltpu.VMEM((LCH, B128), jnp.int16),
        pltpu.VMEM((LCH, DW, B128), jnp.float32),
        pltpu.VMEM((LCH, DW, B128), jnp.float32),
        pltpu.SemaphoreType.DMA,
        pltpu.SemaphoreType.DMA,
        pltpu.SemaphoreType.DMA,
        pltpu.SemaphoreType.DMA,
        pltpu.SemaphoreType.DMA,
        pltpu.SemaphoreType.DMA,
    ]

    @functools.partial(
        pl.kernel,
        out_type=jax.ShapeDtypeStruct((Ldim, DT, BT, D8, B128), jnp.float32),
        mesh=mesh,
        scratch_types=scratch,
        compiler_params=pltpu.CompilerParams(
            use_tc_tiling_on_sc=False, needs_layout_passes=False),
    )
    def k(rt5, it2, out5, stgA, stgB, idxA, idxB, outA, outB,
          stgAs, stgBs, idxAs, idxBs, outAs, outBs):
        cc = lax.axis_index("c")
        ss = lax.axis_index("s")
        dt = ss // 2
        spair = ss % 2
        iota = lax.iota(jnp.int32, LANES)
        bvecs = [j * LANES + iota for j in range(B128 // LANES)]
        djvecs = [jnp.zeros((LANES,), jnp.int32) + dj for dj in range(DW)]

        def stg_src(pair, d80):
            bt = cc * npairs + pair
            return rt5.at[:, dt, bt, pl.ds(d80, DW), :]

        def idx_src(bt, chunk_l0):
            return it2.at[pl.ds(chunk_l0, LCH), pl.ds(bt * B128, B128)]

        def out_dst(bt, d80, l0):
            return out5.at[pl.ds(l0, LCH), dt, bt, pl.ds(d80, DW), :]

        def compute_chunk(stg, idxbuf, outbuf):
            nj = B128 // LANES

            @plsc.parallel_loop(0, LCH, step=1, unroll=5)
            def _body(li):
                gs = {}
                for m in range(nj // 2):
                    packed = plsc.bitcast(
                        idxbuf[li, pl.ds(m * 2 * LANES, 2 * LANES)], jnp.int32)
                    gs[2 * m] = packed & 0xFFFF
                    gs[2 * m + 1] = lax.shift_right_logical(packed, 16)
                vals = [(j, dj,
                         plsc.load_gather(stg, [gs[j], djvecs[dj], bvecs[j]]))
                        for j in range(nj) for dj in range(DW)]
                for j, dj, v in vals:
                    outbuf[li, dj, pl.ds(j * LANES, LANES)] = v

        def unit_compute(pair, bt, d80, stg, next_bt, next_valid, is_first_unit):
            def body(t, carry):
                for kpar, (idxbuf, isem, outbuf, osem) in enumerate(
                        [(idxA, idxAs, outA, outAs), (idxB, idxBs, outB, outBs)]):
                    kchunk = 2 * t + kpar
                    l0 = kchunk * LCH
                    pltpu.make_async_copy(idx_src(bt, l0), idxbuf, isem).wait()
                    if is_first_unit:
                        skip = jnp.logical_and(pair == 0, t == 0)
                    else:
                        skip = jnp.logical_and(pair < 0, t == 0)

                    @pl.when(jnp.logical_not(skip))
                    def _():
                        pltpu.make_async_copy(outbuf, out_dst(bt, d80, l0), osem).wait()

                    compute_chunk(stg, idxbuf, outbuf)
                    pltpu.async_copy(outbuf, out_dst(bt, d80, l0), osem)

                    @pl.when(t < nch // 2 - 1)
                    def _():
                        pltpu.async_copy(idx_src(bt, l0 + 2 * LCH), idxbuf, isem)

                    @pl.when(jnp.logical_and(t == nch // 2 - 1, next_valid))
                    def _():
                        pltpu.async_copy(idx_src(next_bt, kpar * LCH), idxbuf, isem)
                return carry

            lax.fori_loop(0, nch // 2, body, 0)

        d8A = spair * 2 * DW
        d8B = (spair * 2 + 1) * DW
        pltpu.async_copy(stg_src(0, d8A), stgA, stgAs)
        pltpu.async_copy(stg_src(0, d8B), stgB, stgBs)
        bt0 = cc * npairs
        pltpu.async_copy(idx_src(bt0, 0), idxA, idxAs)
        pltpu.async_copy(idx_src(bt0, LCH), idxB, idxBs)

        def pair_body(pair, carry):
            bt = cc * npairs + pair
            has_next = pair + 1 < npairs
            pltpu.make_async_copy(stg_src(pair, d8A), stgA, stgAs).wait()
            unit_compute(pair, bt, d8A, stgA,
                         next_bt=bt, next_valid=jnp.bool_(True),
                         is_first_unit=True)

            @pl.when(has_next)
            def _():
                pltpu.async_copy(stg_src(pair + 1, d8A), stgA, stgAs)

            pltpu.make_async_copy(stg_src(pair, d8B), stgB, stgBs).wait()
            unit_compute(pair, bt, d8B, stgB,
                         next_bt=bt + 1, next_valid=has_next,
                         is_first_unit=False)

            @pl.when(has_next)
            def _():
                pltpu.async_copy(stg_src(pair + 1, d8B), stgB, stgBs)

            return carry

        lax.fori_loop(0, npairs, pair_body, 0)

        lastb = cc * npairs + npairs - 1
        l_last = (nch - 2) * LCH
        pltpu.make_async_copy(outA, out_dst(lastb, d8B, l_last), outAs).wait()
        pltpu.make_async_copy(outB, out_dst(lastb, d8B, l_last + LCH), outBs).wait()

    return k


def kernel(reps, index):
    B, L, D = reps.shape
    DT, D8, B128 = D // 8, 8, 128
    BT = B // B128
    rt = jnp.transpose(reps, (1, 2, 0))
    rt5 = rt.reshape(L, DT, D8, BT, B128)
    rt5 = jnp.transpose(rt5, (0, 1, 3, 2, 4))
    it2 = jnp.transpose(index.reshape(B, L).astype(jnp.int16), (1, 0))
    it2p = (it2.reshape(L, B // 32, 2, LANES)
            .transpose(0, 1, 3, 2)
            .reshape(L, B))
    out5 = _make_k(L, DT, BT, D8, B128)(rt5, it2p)
    out = jnp.transpose(out5, (0, 1, 3, 2, 4)).reshape(L, D, B)
    return jnp.transpose(out, (2, 0, 1))
